# baseline (speedup 1.0000x reference)
import functools

import jax
import jax.numpy as jnp
from jax import lax
from jax.experimental import pallas as pl
from jax.experimental.pallas import tpu as pltpu
from jax.experimental.pallas import tpu_sc as plsc

N = 512
NC2 = N * (N - 1) // 2
B = 128

SC_CORES = 2
SC_SUBCORES = 16
NW = SC_CORES * SC_SUBCORES
BATCH_PER_W = B // NW

RB = 32
NBLK = N // RB

def _tri(i):
    return (i * (i - 1)) // 2

_A = []
_L = []
for _k in range(NBLK):
    a = (_tri(RB * _k) // 8) * 8
    end = _tri(RB * (_k + 1))
    l = -(-(end - a) // 8) * 8
    _A.append(a)
    _L.append(l)
CHUNK_MAX = max(_L) + N + 16

_W128 = [min(N, -(-(RB * (_k + 1)) // 128) * 128) for _k in range(NBLK)]


def _sc_build_body(flat_hbm, p_hbm, chunk0, chunk1, strip0, strip1,
                   cs0, cs1, ss0, ss1):
    wid = lax.axis_index("s") * SC_CORES + lax.axis_index("c")
    lane = lax.iota(jnp.int32, 16)
    chunks = [chunk0, chunk1]
    strips = [strip0, strip1]
    csem = [cs0, cs1]
    ssem = [ss0, ss1]

    def per_batch(bb, carry):
        b = wid * BATCH_PER_W + bb

        def chunk_start(k, slot):
            src_off = pl.multiple_of(b * NC2 + _A[k], 8)
            return pltpu.async_copy(flat_hbm.at[pl.ds(src_off, _L[k])],
                                    chunks[slot].at[pl.ds(0, _L[k])],
                                    csem[slot])

        h = chunk_start(0, 0)
        pending = [None, None]
        for k in range(NBLK):
            cur = k & 1
            hc = h
            if k + 1 < NBLK:
                h = chunk_start(k + 1, 1 - cur)
            hc.wait()
            if pending[cur] is not None:
                pending[cur].wait()
            w_k = RB * (k + 1)

            def per_row(r, c2, k=k, w_k=w_k, cur=cur):
                i = RB * k + r
                off = (i * (i - 1)) // 2 - _A[k]
                for g in range(w_k // 16):
                    idx = off + g * 16 + lane
                    v = plsc.load_gather(chunks[cur], [idx])
                    strips[cur][r, pl.ds(g * 16, 16)] = v
                return c2

            lax.fori_loop(0, RB, per_row, 0)
            pending[cur] = pltpu.async_copy(
                strips[cur].at[pl.ds(0, RB), pl.ds(0, _W128[k])],
                p_hbm.at[b, pl.ds(RB * k, RB), pl.ds(0, _W128[k])],
                ssem[cur])
        for ps in pending:
            if ps is not None:
                ps.wait()
        return carry

    lax.fori_loop(0, BATCH_PER_W, per_batch, 0)


@functools.cache
def _sc_build():
    return pl.kernel(
        _sc_build_body,
        mesh=plsc.VectorSubcoreMesh(core_axis_name="c", subcore_axis_name="s"),
        out_type=jax.ShapeDtypeStruct((B, N, N), jnp.float32),
        scratch_types=[
            pltpu.VMEM((CHUNK_MAX,), jnp.float32),
            pltpu.VMEM((CHUNK_MAX,), jnp.float32),
            pltpu.VMEM((RB, N), jnp.float32),
            pltpu.VMEM((RB, N), jnp.float32),
            pltpu.SemaphoreType.DMA,
            pltpu.SemaphoreType.DMA,
            pltpu.SemaphoreType.DMA,
            pltpu.SemaphoreType.DMA,
        ],
        compiler_params=pltpu.CompilerParams(needs_layout_passes=False),
    )


STRIP = 128
NSTRIP = N // STRIP


BB = 16


def _sym_body(r_ref, o_ref):
    bi = pl.program_id(1)
    bj = pl.program_id(2)
    ig = jax.lax.broadcasted_iota(jnp.int32, (STRIP, STRIP), 0) + bi * STRIP
    jg = jax.lax.broadcasted_iota(jnp.int32, (STRIP, STRIP), 1) + bj * STRIP
    r = r_ref[...]
    rt = jnp.swapaxes(r, 1, 2)
    o_ref[...] = (jnp.where((jg < ig)[None], r, 0.0)
                  + jnp.where((ig < jg)[None], rt, 0.0))


def _sym_call(p, interpret=False):
    b = p.shape[0]
    return pl.pallas_call(
        _sym_body,
        grid=(b // BB, NSTRIP, NSTRIP),
        in_specs=[
            pl.BlockSpec(
                (BB, STRIP, STRIP),
                lambda i, bi, bj: (i, jnp.maximum(bi, bj), jnp.minimum(bi, bj)),
            ),
        ],
        out_specs=pl.BlockSpec((BB, STRIP, STRIP), lambda i, bi, bj: (i, bi, bj)),
        out_shape=jax.ShapeDtypeStruct((b, N, N), jnp.float32),
        interpret=interpret,
    )(p)


C4W = 128
NC4 = N // C4W

_A4 = []
_L4 = []
for _c in range(NC4):
    a = (_tri(C4W * _c) // 8) * 8
    l = -(-(_tri(C4W * (_c + 1)) - a) // 8) * 8
    _A4.append(a)
    _L4.append(l)
C4_MAX = max(_L4) + 144


def _sc_full_body(flat_hbm, q_hbm, c_v, d0, d1, p0, p1,
                  csm, dsm0, dsm1, psm0, psm1):
    wid = lax.axis_index("s") * SC_CORES + lax.axis_index("c")
    lane = lax.iota(jnp.int32, 16)
    zeros16 = jnp.zeros((16,), jnp.float32)
    dbuf = [d0, d1]
    pbuf = [p0, p1]
    dsem = [dsm0, dsm1]
    psem = [psm0, psm1]

    def per_batch(bb, carry):
        b = wid * BATCH_PER_W + bb
        pend_d = [None, None]
        pend_p = [None, None]
        hC = None
        for c4 in range(NC4):
            if hC is None:
                src = pl.multiple_of(b * NC2 + _A4[c4], 8)
                hC = pltpu.async_copy(flat_hbm.at[pl.ds(src, _L4[c4])],
                                      c_v.at[pl.ds(0, _L4[c4])], csm)
            hC.wait()
            base = C4W * c4
            ds = c4 & 1
            if pend_d[ds] is not None:
                pend_d[ds].wait()
            D = dbuf[ds]

            def d_row(r, cr, c4=c4, base=base, D=D):
                i = base + r
                off = (i * (i - 1)) // 2 - _A4[c4] + base
                for g in range(C4W // 16):
                    v = plsc.load_gather(c_v, [off + g * 16 + lane])
                    D[r, pl.ds(g * 16, 16)] = v
                return cr

            lax.fori_loop(0, C4W, d_row, 0)

            def d_col(j, cr, c4=c4, base=base, D=D):
                offj = ((base + j) * (base + j - 1)) // 2 - _A4[c4] + base
                jj = jnp.full((16,), j, jnp.int32)
                for g in range(C4W // 16):
                    rid = g * 16 + lane
                    v = plsc.load_gather(c_v, [offj + rid])
                    plsc.store_scatter(D, [rid, jj], v, mask=rid < j)
                return cr

            lax.fori_loop(0, C4W, d_col, 0)

            for g in range(C4W // 16):
                t = g * 16 + lane
                plsc.store_scatter(D, [t, t], zeros16)

            pend_d[ds] = pltpu.async_copy(
                D, q_hbm.at[b, pl.ds(base, C4W), pl.ds(base, C4W)], dsem[ds])

            for p in range(c4):
                ps = p & 1
                if pend_p[ps] is not None:
                    pend_p[ps].wait()
                PB = pbuf[ps]

                def r_row(r, cr, c4=c4, base=base, p=p, PB=PB):
                    i = base + r
                    off = (i * (i - 1)) // 2 - _A4[c4] + C4W * p
                    for g in range(C4W // 16):
                        v = plsc.load_gather(c_v, [off + g * 16 + lane])
                        PB[r, pl.ds(g * 16, 16)] = v
                    return cr

                lax.fori_loop(0, C4W, r_row, 0)
                pend_p[ps] = pltpu.async_copy(
                    PB, q_hbm.at[b, pl.ds(base, C4W), pl.ds(C4W * p, C4W)],
                    psem[ps])

            for p in range(c4):
                ps = p & 1
                if pend_p[ps] is not None:
                    pend_p[ps].wait()
                PB = pbuf[ps]

                def c_col(j, cr, c4=c4, base=base, p=p, PB=PB):
                    offj = ((base + j) * (base + j - 1)) // 2 - _A4[c4] + C4W * p
                    jj = jnp.full((16,), j, jnp.int32)
                    for g in range(C4W // 16):
                        rid = g * 16 + lane
                        v = plsc.load_gather(c_v, [offj + rid])
                        plsc.store_scatter(PB, [rid, jj], v)
                    return cr

                lax.fori_loop(0, C4W, c_col, 0)
                pend_p[ps] = pltpu.async_copy(
                    PB, q_hbm.at[b, pl.ds(C4W * p, C4W), pl.ds(base, C4W)],
                    psem[ps])

            if c4 + 1 < NC4:
                src = pl.multiple_of(b * NC2 + _A4[c4 + 1], 8)
                hC = pltpu.async_copy(flat_hbm.at[pl.ds(src, _L4[c4 + 1])],
                                      c_v.at[pl.ds(0, _L4[c4 + 1])], csm)
            else:
                hC = None
        for h in pend_d + pend_p:
            if h is not None:
                h.wait()
        return carry

    lax.fori_loop(0, BATCH_PER_W, per_batch, 0)


@functools.cache
def _sc_full():
    return pl.kernel(
        _sc_full_body,
        mesh=plsc.VectorSubcoreMesh(core_axis_name="c", subcore_axis_name="s"),
        out_type=jax.ShapeDtypeStruct((B, N, N), jnp.float32),
        scratch_types=[
            pltpu.VMEM((C4_MAX,), jnp.float32),
            pltpu.VMEM((C4W, C4W), jnp.float32),
            pltpu.VMEM((C4W, C4W), jnp.float32),
            pltpu.VMEM((C4W, C4W), jnp.float32),
            pltpu.VMEM((C4W, C4W), jnp.float32),
            pltpu.SemaphoreType.DMA,
            pltpu.SemaphoreType.DMA,
            pltpu.SemaphoreType.DMA,
            pltpu.SemaphoreType.DMA,
            pltpu.SemaphoreType.DMA,
        ],
        compiler_params=pltpu.CompilerParams(needs_layout_passes=False),
    )


def kernel(decompFE):
    return _sc_full()(decompFE.reshape(-1))


# speedup vs baseline: 1.9678x; 1.9678x over previous
import functools

import jax
import jax.numpy as jnp
from jax import lax
from jax.experimental import pallas as pl
from jax.experimental.pallas import tpu as pltpu
from jax.experimental.pallas import tpu_sc as plsc

N = 512
NC2 = N * (N - 1) // 2
B = 128

SC_CORES = 2
SC_SUBCORES = 16
NW = SC_CORES * SC_SUBCORES
BATCH_PER_W = B // NW

RB = 32
NBLK = N // RB

def _tri(i):
    return (i * (i - 1)) // 2

_A = []
_L = []
for _k in range(NBLK):
    a = (_tri(RB * _k) // 8) * 8
    end = _tri(RB * (_k + 1))
    l = -(-(end - a) // 8) * 8
    _A.append(a)
    _L.append(l)
CHUNK_MAX = max(_L) + N + 16

_W128 = [min(N, -(-(RB * (_k + 1)) // 128) * 128) for _k in range(NBLK)]


def _sc_build_body(flat_hbm, p_hbm, chunk0, chunk1, strip0, strip1,
                   cs0, cs1, ss0, ss1):
    wid = lax.axis_index("s") * SC_CORES + lax.axis_index("c")
    lane = lax.iota(jnp.int32, 16)
    chunks = [chunk0, chunk1]
    strips = [strip0, strip1]
    csem = [cs0, cs1]
    ssem = [ss0, ss1]

    def per_batch(bb, carry):
        b = wid * BATCH_PER_W + bb

        def chunk_start(k, slot):
            src_off = pl.multiple_of(b * NC2 + _A[k], 8)
            return pltpu.async_copy(flat_hbm.at[pl.ds(src_off, _L[k])],
                                    chunks[slot].at[pl.ds(0, _L[k])],
                                    csem[slot])

        h = chunk_start(0, 0)
        pending = [None, None]
        for k in range(NBLK):
            cur = k & 1
            hc = h
            if k + 1 < NBLK:
                h = chunk_start(k + 1, 1 - cur)
            hc.wait()
            if pending[cur] is not None:
                pending[cur].wait()
            w_k = RB * (k + 1)

            def per_row(r, c2, k=k, w_k=w_k, cur=cur):
                i = RB * k + r
                off = (i * (i - 1)) // 2 - _A[k]
                for g in range(w_k // 16):
                    idx = off + g * 16 + lane
                    v = plsc.load_gather(chunks[cur], [idx])
                    strips[cur][r, pl.ds(g * 16, 16)] = v
                return c2

            lax.fori_loop(0, RB, per_row, 0)
            pending[cur] = pltpu.async_copy(
                strips[cur].at[pl.ds(0, RB), pl.ds(0, _W128[k])],
                p_hbm.at[b, pl.ds(RB * k, RB), pl.ds(0, _W128[k])],
                ssem[cur])
        for ps in pending:
            if ps is not None:
                ps.wait()
        return carry

    lax.fori_loop(0, BATCH_PER_W, per_batch, 0)


@functools.cache
def _sc_build():
    return pl.kernel(
        _sc_build_body,
        mesh=plsc.VectorSubcoreMesh(core_axis_name="c", subcore_axis_name="s"),
        out_type=jax.ShapeDtypeStruct((B, N, N), jnp.float32),
        scratch_types=[
            pltpu.VMEM((CHUNK_MAX,), jnp.float32),
            pltpu.VMEM((CHUNK_MAX,), jnp.float32),
            pltpu.VMEM((RB, N), jnp.float32),
            pltpu.VMEM((RB, N), jnp.float32),
            pltpu.SemaphoreType.DMA,
            pltpu.SemaphoreType.DMA,
            pltpu.SemaphoreType.DMA,
            pltpu.SemaphoreType.DMA,
        ],
        compiler_params=pltpu.CompilerParams(needs_layout_passes=False),
    )


STRIP = 128
NSTRIP = N // STRIP


BB = 32


def _sym_body(r_ref, o_ref):
    bi = pl.program_id(1)
    bj = pl.program_id(2)
    ig = jax.lax.broadcasted_iota(jnp.int32, (STRIP, STRIP), 0) + bi * STRIP
    jg = jax.lax.broadcasted_iota(jnp.int32, (STRIP, STRIP), 1) + bj * STRIP
    r = r_ref[...]
    rt = jnp.swapaxes(r, 1, 2)
    o_ref[...] = (jnp.where((jg < ig)[None], r, 0.0)
                  + jnp.where((ig < jg)[None], rt, 0.0))


def _sym_call(p, interpret=False):
    b = p.shape[0]
    return pl.pallas_call(
        _sym_body,
        grid=(b // BB, NSTRIP, NSTRIP),
        in_specs=[
            pl.BlockSpec(
                (BB, STRIP, STRIP),
                lambda i, bi, bj: (i, jnp.maximum(bi, bj), jnp.minimum(bi, bj)),
            ),
        ],
        out_specs=pl.BlockSpec((BB, STRIP, STRIP), lambda i, bi, bj: (i, bi, bj)),
        out_shape=jax.ShapeDtypeStruct((b, N, N), jnp.float32),
        interpret=interpret,
    )(p)


def kernel(decompFE):
    p = _sc_build()(decompFE.reshape(-1))
    return _sym_call(p)


# speedup vs baseline: 2.0654x; 1.0496x over previous
import functools

import jax
import jax.numpy as jnp
from jax import lax
from jax.experimental import pallas as pl
from jax.experimental.pallas import tpu as pltpu
from jax.experimental.pallas import tpu_sc as plsc

N = 512
NC2 = N * (N - 1) // 2
B = 128

SC_CORES = 2
SC_SUBCORES = 16
NW = SC_CORES * SC_SUBCORES
BATCH_PER_W = B // NW

RB = 32
NBLK = N // RB

def _tri(i):
    return (i * (i - 1)) // 2

_A = []
_L = []
for _k in range(NBLK):
    a = (_tri(RB * _k) // 8) * 8
    end = _tri(RB * (_k + 1))
    l = -(-(end - a) // 8) * 8
    _A.append(a)
    _L.append(l)
CHUNK_MAX = max(_L) + N + 16

_W128 = [min(N, -(-(RB * (_k + 1)) // 128) * 128) for _k in range(NBLK)]


def _sc_build_body(flat_hbm, p_hbm, chunk0, chunk1, strip0, strip1,
                   cs0, cs1, ss0, ss1):
    wid = lax.axis_index("s") * SC_CORES + lax.axis_index("c")
    lane = lax.iota(jnp.int32, 16)
    chunks = [chunk0, chunk1]
    strips = [strip0, strip1]
    csem = [cs0, cs1]
    ssem = [ss0, ss1]

    def per_batch(bb, carry):
        b = wid * BATCH_PER_W + bb

        def chunk_start(k, slot):
            src_off = pl.multiple_of(b * NC2 + _A[k], 8)
            return pltpu.async_copy(flat_hbm.at[pl.ds(src_off, _L[k])],
                                    chunks[slot].at[pl.ds(0, _L[k])],
                                    csem[slot])

        h = chunk_start(0, 0)
        pending = [None, None]
        for k in range(NBLK):
            cur = k & 1
            hc = h
            if k + 1 < NBLK:
                h = chunk_start(k + 1, 1 - cur)
            hc.wait()
            if pending[cur] is not None:
                pending[cur].wait()
            w_k = RB * (k + 1)

            def per_row(r, c2, k=k, w_k=w_k, cur=cur):
                i = RB * k + r
                off = (i * (i - 1)) // 2 - _A[k]
                for g in range(w_k // 16):
                    idx = off + g * 16 + lane
                    v = plsc.load_gather(chunks[cur], [idx])
                    strips[cur][r, pl.ds(g * 16, 16)] = v
                return c2

            lax.fori_loop(0, RB, per_row, 0)
            pending[cur] = pltpu.async_copy(
                strips[cur].at[pl.ds(0, RB), pl.ds(0, _W128[k])],
                p_hbm.at[b, pl.ds(RB * k, RB), pl.ds(0, _W128[k])],
                ssem[cur])
        for ps in pending:
            if ps is not None:
                ps.wait()
        return carry

    lax.fori_loop(0, BATCH_PER_W, per_batch, 0)


@functools.cache
def _sc_build():
    return pl.kernel(
        _sc_build_body,
        mesh=plsc.VectorSubcoreMesh(core_axis_name="c", subcore_axis_name="s"),
        out_type=jax.ShapeDtypeStruct((B, N, N), jnp.float32),
        scratch_types=[
            pltpu.VMEM((CHUNK_MAX,), jnp.float32),
            pltpu.VMEM((CHUNK_MAX,), jnp.float32),
            pltpu.VMEM((RB, N), jnp.float32),
            pltpu.VMEM((RB, N), jnp.float32),
            pltpu.SemaphoreType.DMA,
            pltpu.SemaphoreType.DMA,
            pltpu.SemaphoreType.DMA,
            pltpu.SemaphoreType.DMA,
        ],
        compiler_params=pltpu.CompilerParams(needs_layout_passes=False),
    )


STRIP = 128
NSTRIP = N // STRIP


BB = 64


def _sym_body(r_ref, o_ref):
    bi = pl.program_id(1)
    bj = pl.program_id(2)
    ig = jax.lax.broadcasted_iota(jnp.int32, (STRIP, STRIP), 0) + bi * STRIP
    jg = jax.lax.broadcasted_iota(jnp.int32, (STRIP, STRIP), 1) + bj * STRIP
    r = r_ref[...]
    rt = jnp.swapaxes(r, 1, 2)
    o_ref[...] = (jnp.where((jg < ig)[None], r, 0.0)
                  + jnp.where((ig < jg)[None], rt, 0.0))


def _sym_call(p, interpret=False):
    b = p.shape[0]
    return pl.pallas_call(
        _sym_body,
        grid=(b // BB, NSTRIP, NSTRIP),
        in_specs=[
            pl.BlockSpec(
                (BB, STRIP, STRIP),
                lambda i, bi, bj: (i, jnp.maximum(bi, bj), jnp.minimum(bi, bj)),
            ),
        ],
        out_specs=pl.BlockSpec((BB, STRIP, STRIP), lambda i, bi, bj: (i, bi, bj)),
        out_shape=jax.ShapeDtypeStruct((b, N, N), jnp.float32),
        interpret=interpret,
    )(p)


def kernel(decompFE):
    p = _sc_build()(decompFE.reshape(-1))
    return _sym_call(p)


# speedup vs baseline: 2.0893x; 1.0116x over previous
import functools

import jax
import jax.numpy as jnp
from jax import lax
from jax.experimental import pallas as pl
from jax.experimental.pallas import tpu as pltpu
from jax.experimental.pallas import tpu_sc as plsc

N = 512
NC2 = N * (N - 1) // 2
B = 128

SC_CORES = 2
SC_SUBCORES = 16
NW = SC_CORES * SC_SUBCORES
BATCH_PER_W = B // NW

RB = 32
NBLK = N // RB

def _tri(i):
    return (i * (i - 1)) // 2

_A = []
_L = []
for _k in range(NBLK):
    a = (_tri(RB * _k) // 8) * 8
    end = _tri(RB * (_k + 1))
    l = -(-(end - a) // 8) * 8
    _A.append(a)
    _L.append(l)
CHUNK_MAX = max(_L) + N + 16

_W128 = [min(N, -(-(RB * (_k + 1)) // 128) * 128) for _k in range(NBLK)]


def _sc_build_body(flat_hbm, p_hbm, chunk0, chunk1, strip0, strip1,
                   cs0, cs1, ss0, ss1):
    wid = lax.axis_index("s") * SC_CORES + lax.axis_index("c")
    lane = lax.iota(jnp.int32, 16)
    chunks = [chunk0, chunk1]
    strips = [strip0, strip1]
    csem = [cs0, cs1]
    ssem = [ss0, ss1]

    def per_batch(bb, carry):
        b = wid * BATCH_PER_W + bb

        def chunk_start(k, slot):
            src_off = pl.multiple_of(b * NC2 + _A[k], 8)
            return pltpu.async_copy(flat_hbm.at[pl.ds(src_off, _L[k])],
                                    chunks[slot].at[pl.ds(0, _L[k])],
                                    csem[slot])

        h = chunk_start(0, 0)
        pending = [None, None]
        for k in range(NBLK):
            cur = k & 1
            hc = h
            if k + 1 < NBLK:
                h = chunk_start(k + 1, 1 - cur)
            hc.wait()
            if pending[cur] is not None:
                pending[cur].wait()
            w_k = RB * (k + 1)

            def per_row(r, c2, k=k, w_k=w_k, cur=cur):
                i = RB * k + r
                off = (i * (i - 1)) // 2 - _A[k]
                for g in range(w_k // 16):
                    idx = off + g * 16 + lane
                    v = plsc.load_gather(chunks[cur], [idx])
                    strips[cur][r, pl.ds(g * 16, 16)] = v
                return c2

            lax.fori_loop(0, RB, per_row, 0)
            pending[cur] = pltpu.async_copy(
                strips[cur].at[pl.ds(0, RB), pl.ds(0, _W128[k])],
                p_hbm.at[b, pl.ds(RB * k, RB), pl.ds(0, _W128[k])],
                ssem[cur])
        for ps in pending:
            if ps is not None:
                ps.wait()
        return carry

    lax.fori_loop(0, BATCH_PER_W, per_batch, 0)


@functools.cache
def _sc_build():
    return pl.kernel(
        _sc_build_body,
        mesh=plsc.VectorSubcoreMesh(core_axis_name="c", subcore_axis_name="s"),
        out_type=jax.ShapeDtypeStruct((B, N, N), jnp.float32),
        scratch_types=[
            pltpu.VMEM((CHUNK_MAX,), jnp.float32),
            pltpu.VMEM((CHUNK_MAX,), jnp.float32),
            pltpu.VMEM((RB, N), jnp.float32),
            pltpu.VMEM((RB, N), jnp.float32),
            pltpu.SemaphoreType.DMA,
            pltpu.SemaphoreType.DMA,
            pltpu.SemaphoreType.DMA,
            pltpu.SemaphoreType.DMA,
        ],
        compiler_params=pltpu.CompilerParams(needs_layout_passes=False),
    )


STRIP = 128
NSTRIP = N // STRIP


BB = 128


def _sym_body(r_ref, o_ref):
    bi = pl.program_id(1)
    bj = pl.program_id(2)
    ig = jax.lax.broadcasted_iota(jnp.int32, (STRIP, STRIP), 0) + bi * STRIP
    jg = jax.lax.broadcasted_iota(jnp.int32, (STRIP, STRIP), 1) + bj * STRIP
    r = r_ref[...]
    rt = jnp.swapaxes(r, 1, 2)
    o_ref[...] = (jnp.where((jg < ig)[None], r, 0.0)
                  + jnp.where((ig < jg)[None], rt, 0.0))


def _sym_call(p, interpret=False):
    b = p.shape[0]
    return pl.pallas_call(
        _sym_body,
        grid=(b // BB, NSTRIP, NSTRIP),
        in_specs=[
            pl.BlockSpec(
                (BB, STRIP, STRIP),
                lambda i, bi, bj: (i, jnp.maximum(bi, bj), jnp.minimum(bi, bj)),
            ),
        ],
        out_specs=pl.BlockSpec((BB, STRIP, STRIP), lambda i, bi, bj: (i, bi, bj)),
        out_shape=jax.ShapeDtypeStruct((b, N, N), jnp.float32),
        interpret=interpret,
    )(p)


def kernel(decompFE):
    p = _sc_build()(decompFE.reshape(-1))
    return _sym_call(p)
